# Initial kernel scaffold; baseline (speedup 1.0000x reference)
import functools

import numpy as np
import jax
import jax.numpy as jnp
from jax import lax
from jax.experimental import pallas as pl
from jax.experimental.pallas import tpu as pltpu
from jax.experimental.pallas import tpu_sc as plsc

BATCH = 16
IN_F = 2048
OUT_F = 2048
L = 16
NC, NS = 2, 16
NW = NC * NS
O_PER_W = OUT_F // NW
NCHUNK = IN_F // L

_kb1, _kb2 = jax.random.split(jax.random.key(42))
_U_CONST = np.asarray(jax.random.uniform(_kb1, (OUT_F, IN_F), dtype=jnp.float32))
_RC_CONST = np.asarray(jax.random.randint(_kb2, (OUT_F,), 0, IN_F)).astype(np.int32)

_GATHER_DNUMS = lax.GatherDimensionNumbers(
    offset_dims=(), collapsed_slice_dims=(0,), start_index_map=(0,))


def _vperm(v, idx):
    return lax.gather(v, idx[:, None], _GATHER_DNUMS, (1,),
                      mode=lax.GatherScatterMode.PROMISE_IN_BOUNDS)


_MESH = plsc.VectorSubcoreMesh(core_axis_name="c", subcore_axis_name="s")


@functools.partial(
    pl.kernel,
    out_type=jax.ShapeDtypeStruct((BATCH, OUT_F), jnp.float32),
    mesh=_MESH,
    scratch_types=[
        pltpu.VMEM((BATCH, IN_F), jnp.float32),
        pltpu.VMEM((2, 2 * IN_F), jnp.float32),
        pltpu.VMEM((2, IN_F), jnp.float32),
        pltpu.VMEM((O_PER_W,), jnp.int32),
        pltpu.VMEM((BATCH, O_PER_W), jnp.float32),
        pltpu.VMEM((BATCH * L,), jnp.float32),
        pltpu.SemaphoreType.DMA,
        pltpu.SemaphoreType.DMA,
        pltpu.SemaphoreType.DMA,
        pltpu.SemaphoreType.DMA,
    ],
)
def _edge_sel(x_hbm, pw_hbm, u_hbm, rc_hbm, out_hbm,
              x_v, pw_v, u_v, rc_v, out_v, red_v,
              sem_pw0, sem_pw1, sem_u0, sem_u1):
    wid = lax.axis_index("s") * NC + lax.axis_index("c")
    o_base = pl.multiple_of(wid * O_PER_W, O_PER_W)

    pltpu.sync_copy(x_hbm, x_v)
    pltpu.sync_copy(rc_hbm.at[pl.ds(o_base, O_PER_W)], rc_v)

    sems_pw = (sem_pw0, sem_pw1)
    sems_u = (sem_u0, sem_u1)

    lanes = lax.iota(jnp.int32, L)
    ev_idx = (2 * lanes) & (L - 1)
    od_idx = ev_idx + 1
    sh_idx = (lanes - 8) & (L - 1)
    is_lo = lanes < 8

    def start_row(o_local, slot):
        o = o_base + o_local
        pltpu.async_copy(pw_hbm.at[o], pw_v.at[slot], sems_pw[slot])
        pltpu.async_copy(u_hbm.at[o], u_v.at[slot], sems_u[slot])

    def wait_row(o_local, slot):
        o = o_base + o_local
        pltpu.make_async_copy(pw_hbm.at[o], pw_v.at[slot], sems_pw[slot]).wait()
        pltpu.make_async_copy(u_hbm.at[o], u_v.at[slot], sems_u[slot]).wait()

    def compute_row(o_local, slot):
        def chunk(j, carry):
            accs, maxc = carry
            base2 = j * (2 * L)
            v0 = pw_v[slot, pl.ds(base2, L)]
            v1 = pw_v[slot, pl.ds(base2 + L, L)]
            t_lo = _vperm(v0, od_idx) - _vperm(v0, ev_idx)
            t_hi = _vperm(v1, od_idx) - _vperm(v1, ev_idx)
            t = jnp.where(is_lo, t_lo, _vperm(t_hi, sh_idx))
            a0 = jnp.exp(jnp.minimum(jnp.float32(0.0), -t))
            a1 = jnp.exp(jnp.minimum(jnp.float32(0.0), t))
            p = a1 / (a0 + a1)
            uvec = u_v[slot, pl.ds(j * L, L)]
            c = jnp.where(uvec < p, jnp.float32(1.0), jnp.float32(0.0))
            cinv = jnp.float32(1.0) - c
            maxc = jnp.maximum(maxc, c)
            new_accs = tuple(
                jnp.minimum(accs[b], x_v[b, pl.ds(j * L, L)] + cinv)
                for b in range(BATCH))
            return new_accs, maxc

        inf_vec = jnp.full((L,), jnp.inf, jnp.float32)
        accs0 = tuple(inf_vec for _ in range(BATCH))
        accs, maxc = lax.fori_loop(
            0, NCHUNK, chunk, (accs0, jnp.zeros((L,), jnp.float32)))

        for step in (8, 4, 2, 1):
            perm = (lanes + step) & (L - 1)
            maxc = jnp.maximum(maxc, _vperm(maxc, perm))
            accs = tuple(jnp.minimum(a, _vperm(a, perm)) for a in accs)

        for b in range(BATCH):
            red_v[pl.ds(b * L, L)] = accs[b]
        redvec = plsc.load_gather(red_v, [lanes * L])

        rc_splat = plsc.load_gather(rc_v, [jnp.full((L,), o_local, jnp.int32)])
        fix = plsc.load_gather(x_v, [lanes, rc_splat])
        no_conn = maxc <= jnp.float32(0.0)
        final = jnp.minimum(redvec, jnp.where(no_conn, fix, jnp.inf))

        plsc.store_scatter(
            out_v, [lanes, jnp.full((L,), o_local, jnp.int32)], final)

    start_row(0, 0)

    def outer(g, carry):
        for s in range(2):
            o_local = 2 * g + s

            @pl.when(o_local + 1 < O_PER_W)
            def _():
                start_row(o_local + 1, 1 - s)

            wait_row(o_local, s)
            compute_row(o_local, s)
        return carry

    lax.fori_loop(0, O_PER_W // 2, outer, 0)

    pltpu.sync_copy(out_v, out_hbm.at[:, pl.ds(o_base, O_PER_W)])


def kernel(x, prob_weights):
    pw_flat = prob_weights.reshape(OUT_F, 2 * IN_F)
    u = jnp.asarray(_U_CONST)
    rc = jnp.asarray(_RC_CONST)
    return _edge_sel(x, pw_flat, u, rc)


# speedup vs baseline: 1.4445x; 1.4445x over previous
import functools

import numpy as np
import jax
import jax.numpy as jnp
from jax import lax
from jax.experimental import pallas as pl
from jax.experimental.pallas import tpu as pltpu
from jax.experimental.pallas import tpu_sc as plsc

BATCH = 16
IN_F = 2048
OUT_F = 2048
L = 16
NC, NS = 2, 16
NW = NC * NS
O_PER_W = OUT_F // NW
NCHUNK = IN_F // L

_GATHER_DNUMS = lax.GatherDimensionNumbers(
    offset_dims=(), collapsed_slice_dims=(0,), start_index_map=(0,))


def _vperm(v, idx):
    return lax.gather(v, idx[:, None], _GATHER_DNUMS, (1,),
                      mode=lax.GatherScatterMode.PROMISE_IN_BOUNDS)


_MESH = plsc.VectorSubcoreMesh(core_axis_name="c", subcore_axis_name="s")


@functools.partial(
    pl.kernel,
    out_type=jax.ShapeDtypeStruct((OUT_F, BATCH), jnp.float32),
    mesh=_MESH,
    compiler_params=pltpu.CompilerParams(needs_layout_passes=False),
    scratch_types=[
        pltpu.VMEM((BATCH, IN_F), jnp.float32),
        pltpu.VMEM((2, 2 * IN_F), jnp.float32),
        pltpu.VMEM((2, IN_F), jnp.float32),
        pltpu.VMEM((O_PER_W,), jnp.int32),
        pltpu.VMEM((O_PER_W, BATCH), jnp.float32),
        pltpu.VMEM((BATCH * L,), jnp.float32),
        pltpu.SemaphoreType.DMA,
        pltpu.SemaphoreType.DMA,
        pltpu.SemaphoreType.DMA,
        pltpu.SemaphoreType.DMA,
    ],
)
def _edge_sel(x_hbm, pw_hbm, u_hbm, rc_hbm, out_hbm,
              x_v, pw_v, u_v, rc_v, out_v, red_v,
              sem_pw0, sem_pw1, sem_u0, sem_u1):
    wid = lax.axis_index("s") * NC + lax.axis_index("c")
    o_base = pl.multiple_of(wid * O_PER_W, O_PER_W)

    pltpu.sync_copy(x_hbm, x_v)
    pltpu.sync_copy(rc_hbm.at[pl.ds(o_base, O_PER_W)], rc_v)

    sems_pw = (sem_pw0, sem_pw1)
    sems_u = (sem_u0, sem_u1)

    lanes = lax.iota(jnp.int32, L)
    ev_idx = (2 * lanes) & (L - 1)
    od_idx = ev_idx + 1
    sh_idx = (lanes - 8) & (L - 1)
    is_lo = lanes < 8

    def start_row(o_local, slot):
        o = o_base + o_local
        pltpu.async_copy(pw_hbm.at[o], pw_v.at[slot], sems_pw[slot])
        pltpu.async_copy(u_hbm.at[o], u_v.at[slot], sems_u[slot])

    def wait_row(o_local, slot):
        o = o_base + o_local
        pltpu.make_async_copy(pw_hbm.at[o], pw_v.at[slot], sems_pw[slot]).wait()
        pltpu.make_async_copy(u_hbm.at[o], u_v.at[slot], sems_u[slot]).wait()

    def compute_row(o_local, slot):
        def chunk(j, carry):
            accs, maxc = carry
            base2 = j * (2 * L)
            v0 = pw_v[slot, pl.ds(base2, L)]
            v1 = pw_v[slot, pl.ds(base2 + L, L)]
            t_lo = _vperm(v0, od_idx) - _vperm(v0, ev_idx)
            t_hi = _vperm(v1, od_idx) - _vperm(v1, ev_idx)
            t = jnp.where(is_lo, t_lo, _vperm(t_hi, sh_idx))
            a0 = jnp.exp(jnp.minimum(jnp.float32(0.0), -t))
            a1 = jnp.exp(jnp.minimum(jnp.float32(0.0), t))
            p = a1 / (a0 + a1)
            uvec = u_v[slot, pl.ds(j * L, L)]
            c = jnp.where(uvec < p, jnp.float32(1.0), jnp.float32(0.0))
            cinv = jnp.float32(1.0) - c
            maxc = jnp.maximum(maxc, c)
            new_accs = tuple(
                jnp.minimum(accs[b], x_v[b, pl.ds(j * L, L)] + cinv)
                for b in range(BATCH))
            return new_accs, maxc

        inf_vec = jnp.full((L,), jnp.inf, jnp.float32)
        accs0 = tuple(inf_vec for _ in range(BATCH))
        accs, maxc = lax.fori_loop(
            0, NCHUNK, chunk, (accs0, jnp.zeros((L,), jnp.float32)))

        for step in (8, 4, 2, 1):
            perm = (lanes + step) & (L - 1)
            maxc = jnp.maximum(maxc, _vperm(maxc, perm))
            accs = tuple(jnp.minimum(a, _vperm(a, perm)) for a in accs)

        for b in range(BATCH):
            red_v[pl.ds(b * L, L)] = accs[b]
        redvec = plsc.load_gather(red_v, [lanes * L])

        rc_splat = plsc.load_gather(rc_v, [jnp.full((L,), o_local, jnp.int32)])
        fix = plsc.load_gather(x_v, [lanes, rc_splat])
        no_conn = maxc <= jnp.float32(0.0)
        final = jnp.minimum(redvec, jnp.where(no_conn, fix, jnp.inf))

        plsc.store_scatter(
            out_v, [jnp.full((L,), o_local, jnp.int32), lanes], final)

    start_row(0, 0)

    def outer(g, carry):
        for s in range(2):
            o_local = 2 * g + s

            @pl.when(o_local + 1 < O_PER_W)
            def _():
                start_row(o_local + 1, 1 - s)

            wait_row(o_local, s)
            compute_row(o_local, s)
        return carry

    lax.fori_loop(0, O_PER_W // 2, outer, 0)

    pltpu.sync_copy(out_v, out_hbm.at[pl.ds(o_base, O_PER_W)])


def kernel(x, prob_weights):
    kb1, kb2 = jax.random.split(jax.random.key(42))
    u = jax.random.uniform(kb1, (OUT_F, IN_F), dtype=jnp.float32)
    rc = jax.random.randint(kb2, (OUT_F,), 0, IN_F).astype(jnp.int32)
    pw_flat = prob_weights.reshape(OUT_F, 2 * IN_F)
    out_t = _edge_sel(x, pw_flat, u, rc)
    return out_t.T


# speedup vs baseline: 1.9101x; 1.3223x over previous
import functools

import numpy as np
import jax
import jax.numpy as jnp
from jax import lax
from jax.experimental import pallas as pl
from jax.experimental.pallas import tpu as pltpu
from jax.experimental.pallas import tpu_sc as plsc

BATCH = 16
IN_F = 2048
OUT_F = 2048
L = 16
NC, NS = 2, 16
NW = NC * NS
O_PER_W = OUT_F // NW
NCHUNK = IN_F // L


def _rotl32(x, d):
    return ((x << np.uint32(d)) | (x >> np.uint32(32 - d))).astype(np.uint32)


def _threefry2x32(k1, k2, x1, x2):
    rotations = ((13, 15, 26, 6), (17, 29, 16, 24))
    ks = [np.uint32(k1), np.uint32(k2),
          np.uint32(k1) ^ np.uint32(k2) ^ np.uint32(0x1BD11BDA)]
    x = [(x1 + ks[0]).astype(np.uint32), (x2 + ks[1]).astype(np.uint32)]

    def rounds(x, rots):
        for r in rots:
            x[0] = (x[0] + x[1]).astype(np.uint32)
            x[1] = x[0] ^ _rotl32(x[1], r)
        return x

    for i, rots in enumerate((rotations[0], rotations[1], rotations[0],
                              rotations[1], rotations[0])):
        x = rounds(x, rots)
        x[0] = (x[0] + ks[(i + 1) % 3]).astype(np.uint32)
        x[1] = (x[1] + ks[(i + 2) % 3] + np.uint32(i + 1)).astype(np.uint32)
    return x[0], x[1]


def _iota_2x32(shape):
    flat = np.arange(int(np.prod(shape)), dtype=np.uint64)
    c1 = (flat >> np.uint64(32)).astype(np.uint32).reshape(shape)
    c2 = (flat & np.uint64(0xFFFFFFFF)).astype(np.uint32).reshape(shape)
    return c1, c2


def _np_split(key, num=2):
    c1, c2 = _iota_2x32((num,))
    b1, b2 = _threefry2x32(key[0], key[1], c1, c2)
    return [(b1[i], b2[i]) for i in range(num)]


def _np_random_bits(key, shape):
    c1, c2 = _iota_2x32(shape)
    b1, b2 = _threefry2x32(key[0], key[1], c1, c2)
    return b1 ^ b2


def _np_uniform(key, shape):
    bits = _np_random_bits(key, shape)
    fb = (bits >> np.uint32(9)) | np.uint32(0x3F800000)
    floats = fb.view(np.float32) - np.float32(1.0)
    floats = floats * np.float32(1.0) + np.float32(0.0)
    return np.maximum(np.float32(0.0), floats)


def _np_randint(key, shape, minval, maxval):
    k1, k2 = _np_split(key)
    hi = _np_random_bits(k1, shape)
    lo = _np_random_bits(k2, shape)
    span = np.uint32(maxval - minval)
    mult = np.uint32((((2 ** 16) % int(span)) ** 2) % int(span))
    off = ((hi % span) * mult + lo % span) % span
    return (np.int32(minval) + off.astype(np.int32)).astype(np.int32)


_KB1, _KB2 = _np_split((np.uint32(0), np.uint32(42)))
_U_CONST = _np_uniform(_KB1, (OUT_F, IN_F))
_RC_CONST = _np_randint(_KB2, (OUT_F,), 0, IN_F)

with np.errstate(divide="ignore"):
    _u64 = _U_CONST.astype(np.float64)
    _TH_CONST = (np.log(_u64) - np.log1p(-_u64)).astype(np.float32)


_GATHER_DNUMS = lax.GatherDimensionNumbers(
    offset_dims=(), collapsed_slice_dims=(0,), start_index_map=(0,))


def _vperm(v, idx):
    return lax.gather(v, idx[:, None], _GATHER_DNUMS, (1,),
                      mode=lax.GatherScatterMode.PROMISE_IN_BOUNDS)


_MESH = plsc.VectorSubcoreMesh(
    core_axis_name="c", subcore_axis_name="s", num_cores=NC, num_subcores=NS)


@functools.partial(
    pl.kernel,
    out_type=jax.ShapeDtypeStruct((OUT_F, BATCH), jnp.float32),
    mesh=_MESH,
    compiler_params=pltpu.CompilerParams(needs_layout_passes=False),
    scratch_types=[
        pltpu.VMEM((BATCH, IN_F), jnp.float32),
        pltpu.VMEM((2, 2 * IN_F), jnp.float32),
        pltpu.VMEM((2, 2 * IN_F), jnp.float32),
        pltpu.VMEM((O_PER_W,), jnp.int32),
        pltpu.VMEM((O_PER_W, BATCH), jnp.float32),
        pltpu.VMEM((BATCH * L,), jnp.float32),
        pltpu.SemaphoreType.DMA,
        pltpu.SemaphoreType.DMA,
        pltpu.SemaphoreType.DMA,
        pltpu.SemaphoreType.DMA,
    ],
)
def _edge_sel(x_hbm, t_hbm, th_hbm, rc_hbm, out_hbm,
              x_v, t_v, th_v, rc_v, out_v, red_v,
              sem_t0, sem_t1, sem_th0, sem_th1):
    wid = lax.axis_index("s") * NC + lax.axis_index("c")
    o_base = pl.multiple_of(wid * O_PER_W, O_PER_W)

    pltpu.sync_copy(x_hbm, x_v)
    pltpu.sync_copy(rc_hbm.at[pl.ds(o_base, O_PER_W)], rc_v)

    sems_t = (sem_t0, sem_t1)
    sems_th = (sem_th0, sem_th1)

    lanes = lax.iota(jnp.int32, L)

    def _pair_copies(pair, slot):
        row2 = pl.ds((o_base + 2 * pair) * IN_F, 2 * IN_F)
        return (
            pltpu.make_async_copy(t_hbm.at[row2], t_v.at[slot], sems_t[slot]),
            pltpu.make_async_copy(th_hbm.at[row2], th_v.at[slot], sems_th[slot]),
        )

    def start_pair(pair, slot):
        for cp in _pair_copies(pair, slot):
            cp.start()

    def wait_pair(pair, slot):
        for cp in _pair_copies(pair, slot):
            cp.wait()

    def emit_row(o_local, accs, minc):
        for step in (8, 4, 2, 1):
            perm = (lanes + step) & (L - 1)
            minc = jnp.minimum(minc, _vperm(minc, perm))
            accs = tuple(jnp.minimum(a, _vperm(a, perm)) for a in accs)

        for b in range(BATCH):
            red_v[pl.ds(b * L, L)] = accs[b]
        redvec = plsc.load_gather(red_v, [lanes * L])

        rc_splat = plsc.load_gather(rc_v, [jnp.full((L,), o_local, jnp.int32)])
        fix = plsc.load_gather(x_v, [lanes, rc_splat])
        no_conn = minc > jnp.float32(0.5)
        final = jnp.minimum(redvec, jnp.where(no_conn, fix, jnp.inf))

        plsc.store_scatter(
            out_v, [jnp.full((L,), o_local, jnp.int32), lanes], final)

    def compute_pair(pair, slot):
        def chunk(g, carry):
            accs_a, accs_b, minc_a, minc_b = carry
            accs_a = list(accs_a)
            accs_b = list(accs_b)
            for jj in range(2):
                base = (2 * g + jj) * L
                ta = t_v[slot, pl.ds(base, L)]
                tb = t_v[slot, pl.ds(IN_F + base, L)]
                tha = th_v[slot, pl.ds(base, L)]
                thb = th_v[slot, pl.ds(IN_F + base, L)]
                one = jnp.float32(1.0)
                zero = jnp.float32(0.0)
                cinv_a = jnp.where(ta > tha, zero, one)
                cinv_b = jnp.where(tb > thb, zero, one)
                minc_a = jnp.minimum(minc_a, cinv_a)
                minc_b = jnp.minimum(minc_b, cinv_b)
                for b in range(BATCH):
                    xb = x_v[b, pl.ds(base, L)]
                    accs_a[b] = jnp.minimum(accs_a[b], xb + cinv_a)
                    accs_b[b] = jnp.minimum(accs_b[b], xb + cinv_b)
            return tuple(accs_a), tuple(accs_b), minc_a, minc_b

        inf_vec = jnp.full((L,), jnp.inf, jnp.float32)
        one_vec = jnp.full((L,), 1.0, jnp.float32)
        accs0 = tuple(inf_vec for _ in range(BATCH))
        accs_a, accs_b, minc_a, minc_b = lax.fori_loop(
            0, NCHUNK // 2, chunk, (accs0, accs0, one_vec, one_vec))

        emit_row(2 * pair, accs_a, minc_a)
        emit_row(2 * pair + 1, accs_b, minc_b)

    start_pair(0, 0)
    n_pairs = O_PER_W // 2

    def outer(g, carry):
        for s in range(2):
            pair = 2 * g + s

            @pl.when(pair + 1 < n_pairs)
            def _():
                start_pair(pair + 1, 1 - s)

            wait_pair(pair, s)
            compute_pair(pair, s)
        return carry

    lax.fori_loop(0, n_pairs // 2, outer, 0)

    pltpu.sync_copy(out_v, out_hbm.at[pl.ds(o_base, O_PER_W)])


def kernel(x, prob_weights):
    t = (prob_weights[:, :, 1] - prob_weights[:, :, 0]).reshape(-1)
    th = jnp.asarray(_TH_CONST.reshape(-1))
    rc = jnp.asarray(_RC_CONST)
    out_t = _edge_sel(x, t, th, rc)
    return out_t.T


# speedup vs baseline: 3.0887x; 1.6170x over previous
import functools

import numpy as np
import jax
import jax.numpy as jnp
from jax import lax
from jax.experimental import pallas as pl
from jax.experimental.pallas import tpu as pltpu
from jax.experimental.pallas import tpu_sc as plsc

BATCH = 16
IN_F = 2048
OUT_F = 2048
L = 16
NC, NS = 2, 16
NW = NC * NS
O_PER_W = OUT_F // NW
NCHUNK = IN_F // L


def _rotl32(x, d):
    return ((x << np.uint32(d)) | (x >> np.uint32(32 - d))).astype(np.uint32)


def _threefry2x32(k1, k2, x1, x2):
    rotations = ((13, 15, 26, 6), (17, 29, 16, 24))
    ks = [np.uint32(k1), np.uint32(k2),
          np.uint32(k1) ^ np.uint32(k2) ^ np.uint32(0x1BD11BDA)]
    x = [(x1 + ks[0]).astype(np.uint32), (x2 + ks[1]).astype(np.uint32)]

    def rounds(x, rots):
        for r in rots:
            x[0] = (x[0] + x[1]).astype(np.uint32)
            x[1] = x[0] ^ _rotl32(x[1], r)
        return x

    for i, rots in enumerate((rotations[0], rotations[1], rotations[0],
                              rotations[1], rotations[0])):
        x = rounds(x, rots)
        x[0] = (x[0] + ks[(i + 1) % 3]).astype(np.uint32)
        x[1] = (x[1] + ks[(i + 2) % 3] + np.uint32(i + 1)).astype(np.uint32)
    return x[0], x[1]


def _iota_2x32(shape):
    flat = np.arange(int(np.prod(shape)), dtype=np.uint64)
    c1 = (flat >> np.uint64(32)).astype(np.uint32).reshape(shape)
    c2 = (flat & np.uint64(0xFFFFFFFF)).astype(np.uint32).reshape(shape)
    return c1, c2


def _np_split(key, num=2):
    c1, c2 = _iota_2x32((num,))
    b1, b2 = _threefry2x32(key[0], key[1], c1, c2)
    return [(b1[i], b2[i]) for i in range(num)]


def _np_random_bits(key, shape):
    c1, c2 = _iota_2x32(shape)
    b1, b2 = _threefry2x32(key[0], key[1], c1, c2)
    return b1 ^ b2


def _np_uniform(key, shape):
    bits = _np_random_bits(key, shape)
    fb = (bits >> np.uint32(9)) | np.uint32(0x3F800000)
    floats = fb.view(np.float32) - np.float32(1.0)
    floats = floats * np.float32(1.0) + np.float32(0.0)
    return np.maximum(np.float32(0.0), floats)


def _np_randint(key, shape, minval, maxval):
    k1, k2 = _np_split(key)
    hi = _np_random_bits(k1, shape)
    lo = _np_random_bits(k2, shape)
    span = np.uint32(maxval - minval)
    mult = np.uint32((((2 ** 16) % int(span)) ** 2) % int(span))
    off = ((hi % span) * mult + lo % span) % span
    return (np.int32(minval) + off.astype(np.int32)).astype(np.int32)


_KB1, _KB2 = _np_split((np.uint32(0), np.uint32(42)))
_U_CONST = _np_uniform(_KB1, (OUT_F, IN_F))
_RC_CONST = _np_randint(_KB2, (OUT_F,), 0, IN_F)

with np.errstate(divide="ignore"):
    _u64 = _U_CONST.astype(np.float64)
    _TH_CONST = (np.log(_u64) - np.log1p(-_u64)).astype(np.float32)


_GATHER_DNUMS = lax.GatherDimensionNumbers(
    offset_dims=(), collapsed_slice_dims=(0,), start_index_map=(0,))


def _vperm(v, idx):
    return lax.gather(v, idx[:, None], _GATHER_DNUMS, (1,),
                      mode=lax.GatherScatterMode.PROMISE_IN_BOUNDS)


_MESH = plsc.VectorSubcoreMesh(
    core_axis_name="c", subcore_axis_name="s", num_cores=NC, num_subcores=NS)


@functools.partial(
    pl.kernel,
    out_type=jax.ShapeDtypeStruct((OUT_F, BATCH), jnp.float32),
    mesh=_MESH,
    compiler_params=pltpu.CompilerParams(needs_layout_passes=False),
    scratch_types=[
        pltpu.VMEM((BATCH, IN_F), jnp.float32),
        pltpu.VMEM((2, 2 * IN_F), jnp.float32),
        pltpu.VMEM((2, 2 * IN_F), jnp.float32),
        pltpu.VMEM((O_PER_W,), jnp.int32),
        pltpu.VMEM((O_PER_W, BATCH), jnp.float32),
        pltpu.VMEM((BATCH * L,), jnp.float32),
        pltpu.SemaphoreType.DMA,
        pltpu.SemaphoreType.DMA,
        pltpu.SemaphoreType.DMA,
        pltpu.SemaphoreType.DMA,
    ],
)
def _edge_sel(x_hbm, t_hbm, th_hbm, rc_hbm, out_hbm,
              x_v, t_v, th_v, rc_v, out_v, red_v,
              sem_t0, sem_t1, sem_th0, sem_th1):
    wid = lax.axis_index("s") * NC + lax.axis_index("c")
    o_base = pl.multiple_of(wid * O_PER_W, O_PER_W)

    pltpu.sync_copy(x_hbm, x_v)
    pltpu.sync_copy(rc_hbm.at[pl.ds(o_base, O_PER_W)], rc_v)

    sems_t = (sem_t0, sem_t1)
    sems_th = (sem_th0, sem_th1)

    lanes = lax.iota(jnp.int32, L)

    def _pair_copies(pair, slot):
        row2 = pl.ds((o_base + 2 * pair) * IN_F, 2 * IN_F)
        return (
            pltpu.make_async_copy(t_hbm.at[row2], t_v.at[slot], sems_t[slot]),
            pltpu.make_async_copy(th_hbm.at[row2], th_v.at[slot], sems_th[slot]),
        )

    def start_pair(pair, slot):
        for cp in _pair_copies(pair, slot):
            cp.start()

    def wait_pair(pair, slot):
        for cp in _pair_copies(pair, slot):
            cp.wait()

    def emit_row(o_local, accs, minc):
        for step in (8, 4, 2, 1):
            perm = (lanes + step) & (L - 1)
            minc = jnp.minimum(minc, _vperm(minc, perm))
            accs = tuple(jnp.minimum(a, _vperm(a, perm)) for a in accs)

        for b in range(BATCH):
            red_v[pl.ds(b * L, L)] = accs[b]
        redvec = plsc.load_gather(red_v, [lanes * L])

        rc_splat = plsc.load_gather(rc_v, [jnp.full((L,), o_local, jnp.int32)])
        fix = plsc.load_gather(x_v, [lanes, rc_splat])
        no_conn = minc > jnp.float32(0.5)
        final = jnp.minimum(redvec, jnp.where(no_conn, fix, jnp.inf))

        plsc.store_scatter(
            out_v, [jnp.full((L,), o_local, jnp.int32), lanes], final)

    def compute_pair(pair, slot):
        def chunk(j, carry):
            accs_a, accs_b, minc_a, minc_b = carry
            base = j * L
            ta = t_v[slot, pl.ds(base, L)]
            tb = t_v[slot, pl.ds(IN_F + base, L)]
            tha = th_v[slot, pl.ds(base, L)]
            thb = th_v[slot, pl.ds(IN_F + base, L)]
            one = jnp.float32(1.0)
            zero = jnp.float32(0.0)
            cinv_a = jnp.where(ta > tha, zero, one)
            cinv_b = jnp.where(tb > thb, zero, one)
            minc_a = jnp.minimum(minc_a, cinv_a)
            minc_b = jnp.minimum(minc_b, cinv_b)
            new_a = []
            new_b = []
            for b in range(BATCH):
                xb = x_v[b, pl.ds(base, L)]
                new_a.append(jnp.minimum(accs_a[b], xb + cinv_a))
                new_b.append(jnp.minimum(accs_b[b], xb + cinv_b))
            return tuple(new_a), tuple(new_b), minc_a, minc_b

        inf_vec = jnp.full((L,), jnp.inf, jnp.float32)
        one_vec = jnp.full((L,), 1.0, jnp.float32)
        accs0 = tuple(inf_vec for _ in range(BATCH))
        accs_a, accs_b, minc_a, minc_b = lax.fori_loop(
            0, NCHUNK, chunk, (accs0, accs0, one_vec, one_vec))

        emit_row(2 * pair, accs_a, minc_a)
        emit_row(2 * pair + 1, accs_b, minc_b)

    start_pair(0, 0)
    n_pairs = O_PER_W // 2

    def outer(g, carry):
        for s in range(2):
            pair = 2 * g + s

            @pl.when(pair + 1 < n_pairs)
            def _():
                start_pair(pair + 1, 1 - s)

            wait_pair(pair, s)
            compute_pair(pair, s)
        return carry

    lax.fori_loop(0, n_pairs // 2, outer, 0)

    pltpu.sync_copy(out_v, out_hbm.at[pl.ds(o_base, O_PER_W)])


def kernel(x, prob_weights):
    t = (prob_weights[:, :, 1] - prob_weights[:, :, 0]).reshape(-1)
    th = jnp.asarray(_TH_CONST.reshape(-1))
    rc = jnp.asarray(_RC_CONST)
    out_t = _edge_sel(x, t, th, rc)
    return out_t.T
